# baseline (speedup 1.0000x reference)
import functools

import jax
import jax.numpy as jnp
from jax import lax
from jax.experimental import pallas as pl
from jax.experimental.pallas import tpu as pltpu
from jax.experimental.pallas import tpu_sc as plsc

_NV = 1000001
_B = 16384
_D = 64
_NW = 32
_RNG = 32768
_BLK = 512
_TAILC0 = (_NV // _BLK) * _BLK
_HCH = 2048
_BATCH = 128
_NCTX = _B * 20
_NCN = _B * 6
_PAIR_ROWS = 8448
_DUMP_PAIR = 8192
_CN_ROWS = _NCN + _BATCH
dump_slot = _NCN


def _sload(ref, i):
    return plsc.load_gather(ref, [jnp.full((16,), i, jnp.int32)])[0]


def _hit_walker(sorted_idx, sorted_dst, bv, sbufs, tailv, idxc, dstc,
                nfull, cb0, table, sems, pad_dst, emit_group, flush):
    lane = lax.iota(jnp.int32, 16)

    def make_grp_body(buf, base_col, width, e0, e1):
        def grp_body(i, carry):
            ctr, chunk_lo = carry
            pos0 = (e0 & -16) + i * 16
            need = pos0 + 16 > chunk_lo + _HCH
            new_lo = jnp.where(need, pos0, chunk_lo)

            @pl.when(need)
            def _():
                a0 = pl.multiple_of(pos0 & -16, 8)
                pltpu.sync_copy(sorted_idx.at[pl.ds(a0, _HCH)], idxc)
                pltpu.sync_copy(sorted_dst.at[pl.ds(a0, _HCH)], dstc)

            local = pl.multiple_of(pos0 - new_lo, 8)
            idxv = idxc[pl.ds(local, 16)]
            dstv = dstc[pl.ds(local, 16)]
            lanepos = pos0 + lane
            valid = jnp.logical_and(lanepos >= e0, lanepos < e1)
            c_loc = jnp.clip(idxv - base_col, 0, width - 1)
            dstv = jnp.where(valid, dstv, pad_dst)
            ctr2 = emit_group(ctr, c_loc, dstv, buf)
            do_flush = ctr2 == _BATCH

            @pl.when(do_flush)
            def _():
                flush()

            return (jnp.where(do_flush, 0, ctr2), new_lo)

        return grp_body

    def pair_body(i, carry):
        for par in range(2):
            n = i * 2 + par

            @pl.when(n < nfull)
            def _():
                pltpu.make_async_copy(
                    table.at[:, pl.ds(cb0 + n * _BLK, _BLK)],
                    sbufs[par], sems.at[par]).wait()

            e0 = _sload(bv, n)
            e1 = jnp.where(n < nfull, _sload(bv, n + 1), e0)
            ng = jnp.maximum(0, (e1 - (e0 & -16) + 15) // 16)
            carry = lax.fori_loop(
                0, ng, make_grp_body(sbufs[par], cb0 + n * _BLK, _BLK, e0, e1),
                carry)

            @pl.when(n + 2 < nfull)
            def _():
                pltpu.async_copy(
                    table.at[:, pl.ds(cb0 + (n + 2) * _BLK, _BLK)],
                    sbufs[par], sems.at[par])

        return carry

    for par in range(2):
        @pl.when(par < nfull)
        def _():
            pltpu.async_copy(table.at[:, pl.ds(cb0 + par * _BLK, _BLK)],
                             sbufs[par], sems.at[par])

    carry = lax.fori_loop(0, 32, pair_body, (jnp.int32(0), jnp.int32(-2**30)))
    e_t0 = _sload(bv, nfull)
    e_t1 = _sload(bv, 64)
    ng = jnp.maximum(0, (e_t1 - (e_t0 & -16) + 15) // 16)
    carry = lax.fori_loop(0, ng,
                          make_grp_body(tailv, _TAILC0, 128, e_t0, e_t1),
                          carry)
    return carry[0]


def _make_gc_body(dump_slot):
  def _gc_body(scn_idx, scn_dst, bounds, xt, tailt, rows_out,
               sbuf0, sbuf1, tailv, idxc, dstc, bv, rstage, sidx, sems):
    wid = lax.axis_index("s") * 2 + lax.axis_index("c")
    cb0 = wid * _RNG
    nfull = jnp.clip((_NV - cb0) // _BLK, 0, 64)

    pltpu.sync_copy(tailt, tailv)
    pltpu.sync_copy(bounds.at[pl.ds(pl.multiple_of(wid * 64, 8), 80)], bv)

    lane = lax.iota(jnp.int32, 16)
    zero16 = jnp.zeros((16,), jnp.float32)
    dvecs = [lane + 16 * q for q in range(4)]
    for k in range(8):
        sidx[pl.ds(k * 16, 16)] = jnp.full((16,), dump_slot, jnp.int32)
    for r in range(_BATCH):
        for q in range(4):
            rstage[r, pl.ds(64 + q * 16, 16)] = zero16

    def flush():
        pltpu.async_copy(rstage, rows_out.at[sidx], sems.at[2]).wait()

    def emit_group(ctr, c_loc, dstv, buf):
        plsc.store_scatter(sidx, [ctr + lane], dstv)
        for h in range(16):
            cvec = jnp.full((16,), c_loc[h], jnp.int32)
            for q in range(4):
                rstage[ctr + h, pl.ds(q * 16, 16)] = plsc.load_gather(
                    buf, [dvecs[q], cvec])
        return ctr + 16

    ctr = _hit_walker(scn_idx, scn_dst, bv, (sbuf0, sbuf1), tailv, idxc, dstc,
                      nfull, cb0, xt, sems, dump_slot, emit_group, flush)

    def pad_body(r, _):
        @pl.when(r >= ctr)
        def _():
            plsc.store_scatter(sidx, [jnp.full((16,), r, jnp.int32)],
                               jnp.full((16,), dump_slot, jnp.int32),
                               mask=lane == 0)
        return 0

    lax.fori_loop(0, _BATCH, pad_body, 0)
    flush()

  return _gc_body


def _k3_body(cn_ref, ctx_ref, out_ref):
    ctx = ctx_ref[...].reshape(256, 20, 128)
    csum = jnp.sum(ctx, axis=1).reshape(256, 1, 128)
    cn = cn_ref[...].reshape(256, 6, 128)
    out_ref[...] = jnp.sum(cn * csum, axis=2)


def kernel(x, center_table, context_table):
    xm = (x + _NV) % _NV
    cn = xm[:, :6].reshape(_NCN)
    cx = xm[:, 6:].reshape(_NCTX)

    def prep(idx_flat, dstdiv):
        iota = jnp.arange(idx_flat.shape[0], dtype=jnp.int32)
        sidx, order = lax.sort((idx_flat, iota), num_keys=1)
        bounds = jnp.searchsorted(sidx, jnp.arange(0, 1048577, _BLK,
                                                   dtype=jnp.int32),
                                  ).astype(jnp.int32)
        bounds = jnp.pad(bounds, (0, 2064 - bounds.shape[0]))
        sidx = jnp.pad(sidx, (0, _HCH))
        dst = jnp.pad(order, (0, _HCH))
        return sidx, dst, bounds

    scx_idx, scx_dst, bounds_cx = prep(cx, 20)
    scn_idx, scn_dst, bounds_cn = prep(cn, 6)

    ctx_t = context_table.T
    cen_t = center_table.T
    tail_cx = jnp.pad(ctx_t[:, _TAILC0:], ((0, 0), (0, 128 - (_NV - _TAILC0))))
    tail_cn = jnp.pad(cen_t[:, _TAILC0:], ((0, 0), (0, 128 - (_NV - _TAILC0))))

    mesh = plsc.VectorSubcoreMesh(core_axis_name="c", subcore_axis_name="s")
    cparams = pltpu.CompilerParams(needs_layout_passes=False)

    def gc_kernel(n_rows, dump_slot):
        return pl.kernel(
            _make_gc_body(dump_slot),
            out_type=jax.ShapeDtypeStruct((n_rows, 128), jnp.float32),
            mesh=mesh,
            compiler_params=cparams,
            scratch_types=[
                pltpu.VMEM((64, _BLK), jnp.float32),
                pltpu.VMEM((64, _BLK), jnp.float32),
                pltpu.VMEM((64, 128), jnp.float32),
                pltpu.VMEM((_HCH,), jnp.int32),
                pltpu.VMEM((_HCH,), jnp.int32),
                pltpu.VMEM((80,), jnp.int32),
                pltpu.VMEM((_BATCH, 128), jnp.float32),
                pltpu.VMEM((_BATCH,), jnp.int32),
                pltpu.SemaphoreType.DMA((3,)),
            ],
        )

    ctx_rows = gc_kernel(_NCTX + _BATCH, _NCTX)(
        scx_idx, scx_dst, bounds_cx, ctx_t, tail_cx)
    cn_rows = gc_kernel(_NCN + _BATCH, _NCN)(
        scn_idx, scn_dst, bounds_cn, cen_t, tail_cn)

    k3 = pl.pallas_call(
        _k3_body,
        out_shape=jax.ShapeDtypeStruct((_B, 6), jnp.float32),
        grid=(_B // 256,),
        in_specs=[
            pl.BlockSpec((1536, 128), lambda i: (i, 0)),
            pl.BlockSpec((5120, 128), lambda i: (i, 0)),
        ],
        out_specs=pl.BlockSpec((256, 6), lambda i: (i, 0)),
    )
    scores = k3(cn_rows, ctx_rows)
    return (scores[:, :1], scores[:, 1:])


# speedup vs baseline: 3.0261x; 3.0261x over previous
import jax
import jax.numpy as jnp
from jax import lax
from jax.experimental import pallas as pl
from jax.experimental.pallas import tpu as pltpu
from jax.experimental.pallas import tpu_sc as plsc

_NV = 1000001
_D = 64
_B = 16384
_NW = 32
_RW = _B // _NW
_CB = 32
_NCH = _RW // _CB
_NBUF = 2

_CN_STRIP = 96
_CTX_STRIP = 128
_CN_SPC = (_CB * 6) // _CN_STRIP
_CTX_SPC = (_CB * 20) // _CTX_STRIP


def _ctx_body(ctx_idx, context_hbm, out_hbm,
              idx_0, idx_1, ctx_buf, sum_buf, sems):
    wid = lax.axis_index("s") * 2 + lax.axis_index("c")
    row0 = wid * _RW
    idx0 = wid * (_RW * 20)
    idx_v = (idx_0, idx_1)

    def issue(g, b):
        pltpu.sync_copy(ctx_idx.at[pl.ds(idx0 + g * (20 * _CB), 20 * _CB)],
                        idx_v[b])
        for j in range(_CTX_SPC):
            pltpu.async_copy(
                context_hbm.at[idx_v[b].at[pl.ds(j * _CTX_STRIP, _CTX_STRIP)]],
                ctx_buf.at[b, pl.ds(j * _CTX_STRIP, _CTX_STRIP)],
                sems.at[b])

    def drain(b):
        pltpu.make_async_copy(context_hbm.at[pl.ds(0, 20 * _CB)],
                              ctx_buf.at[b], sems.at[b]).wait()

    def compute(g, b):
        def row_body(r, carry):
            cbase = r * 20
            acc = [ctx_buf[b, cbase, pl.ds(d * 16, 16)] for d in range(4)]
            for t in range(1, 20):
                for d in range(4):
                    acc[d] = acc[d] + ctx_buf[b, cbase + t, pl.ds(d * 16, 16)]
            for d in range(4):
                sum_buf[r, pl.ds(d * 16, 16)] = acc[d]
            return carry

        lax.fori_loop(0, _CB, row_body, 0)
        pltpu.sync_copy(sum_buf, out_hbm.at[pl.ds(row0 + g * _CB, _CB)])

    issue(0, 0)
    issue(1, 1)

    def pair_body(i, carry):
        for b in range(_NBUF):
            g = i * _NBUF + b
            drain(b)
            compute(g, b)

            @pl.when(g + _NBUF < _NCH)
            def _():
                issue(g + _NBUF, b)

        return carry

    lax.fori_loop(0, _NCH // _NBUF, pair_body, 0)


def _cn_body(cn_idx, center_hbm, ctxsum_hbm, out_hbm,
             idx_0, idx_1, cn_buf, cs_buf, score_buf, sems, csem):
    wid = lax.axis_index("s") * 2 + lax.axis_index("c")
    row0 = wid * _RW
    idx0 = wid * (_RW * 6)
    idx_v = (idx_0, idx_1)

    def issue(g, b):
        pltpu.sync_copy(cn_idx.at[pl.ds(idx0 + g * (6 * _CB), 6 * _CB)],
                        idx_v[b])
        for j in range(_CN_SPC):
            pltpu.async_copy(
                center_hbm.at[idx_v[b].at[pl.ds(j * _CN_STRIP, _CN_STRIP)]],
                cn_buf.at[b, pl.ds(j * _CN_STRIP, _CN_STRIP)],
                sems.at[b])
        pltpu.async_copy(ctxsum_hbm.at[pl.ds(row0 + g * _CB, _CB)],
                         cs_buf.at[b], csem.at[b])

    def drain(b):
        pltpu.make_async_copy(center_hbm.at[pl.ds(0, 6 * _CB)],
                              cn_buf.at[b], sems.at[b]).wait()
        pltpu.make_async_copy(ctxsum_hbm.at[pl.ds(0, _CB)],
                              cs_buf.at[b], csem.at[b]).wait()

    lane = lax.iota(jnp.int32, 16)
    lane15 = lane == 15

    def compute(g, b):
        def row_body(r, carry):
            acc = [cs_buf[b, r, pl.ds(d * 16, 16)] for d in range(4)]
            nbase = r * 6
            for k in range(6):
                v = cn_buf[b, nbase + k, pl.ds(0, 16)] * acc[0]
                for d in range(1, 4):
                    v = v + cn_buf[b, nbase + k, pl.ds(d * 16, 16)] * acc[d]
                iv = jnp.full((16,), r * 6 + k, jnp.int32)
                plsc.store_scatter(score_buf, [iv], plsc.cumsum(v),
                                   mask=lane15)
            return carry

        lax.fori_loop(0, _CB, row_body, 0)
        pltpu.sync_copy(score_buf,
                        out_hbm.at[pl.ds((row0 + g * _CB) * 6, _CB * 6)])

    issue(0, 0)
    issue(1, 1)

    def pair_body(i, carry):
        for b in range(_NBUF):
            g = i * _NBUF + b
            drain(b)
            compute(g, b)

            @pl.when(g + _NBUF < _NCH)
            def _():
                issue(g + _NBUF, b)

        return carry

    lax.fori_loop(0, _NCH // _NBUF, pair_body, 0)


def kernel(x, center_table, context_table):
    xm = (x + _NV) % _NV
    cn_idx = xm[:, :6].reshape(_B * 6)
    ctx_idx = xm[:, 6:].reshape(_B * 20)

    mesh = plsc.VectorSubcoreMesh(core_axis_name="c", subcore_axis_name="s")
    cparams = pltpu.CompilerParams(use_tc_tiling_on_sc=False,
                                   needs_layout_passes=False)

    ctx_k = pl.kernel(
        _ctx_body,
        out_type=jax.ShapeDtypeStruct((_B, _D), jnp.float32),
        mesh=mesh,
        compiler_params=cparams,
        scratch_types=[
            pltpu.VMEM((20 * _CB,), jnp.int32),
            pltpu.VMEM((20 * _CB,), jnp.int32),
            pltpu.VMEM((_NBUF, 20 * _CB, _D), jnp.float32),
            pltpu.VMEM((_CB, _D), jnp.float32),
            pltpu.SemaphoreType.DMA((_NBUF,)),
        ],
    )
    ctx_sum = ctx_k(ctx_idx, context_table)

    cn_k = pl.kernel(
        _cn_body,
        out_type=jax.ShapeDtypeStruct((_B * 6,), jnp.float32),
        mesh=mesh,
        compiler_params=cparams,
        scratch_types=[
            pltpu.VMEM((6 * _CB,), jnp.int32),
            pltpu.VMEM((6 * _CB,), jnp.int32),
            pltpu.VMEM((_NBUF, 6 * _CB, _D), jnp.float32),
            pltpu.VMEM((_NBUF, _CB, _D), jnp.float32),
            pltpu.VMEM((_CB * 6,), jnp.float32),
            pltpu.SemaphoreType.DMA((_NBUF,)),
            pltpu.SemaphoreType.DMA((_NBUF,)),
        ],
    )
    out = cn_k(cn_idx, center_table, ctx_sum).reshape(_B, 6)
    return (out[:, :1], out[:, 1:])
